# Initial kernel scaffold; baseline (speedup 1.0000x reference)
import jax
import jax.numpy as jnp
from jax.experimental import pallas as pl


def kernel(inputs):
    raise NotImplementedError("write your pallas kernel here")


# speedup vs baseline: 5.0838x; 5.0838x over previous
import jax
import jax.numpy as jnp
from jax.experimental import pallas as pl


def _stage(x, j, want_desc_fn):
    s, lanes = x.shape
    if j >= 8:
        g = s // (2 * j)
        xr = x.reshape(g, 2, j, lanes)
        top = xr[:, 0]
        bot = xr[:, 1]
        r0 = jax.lax.broadcasted_iota(jnp.int32, (g, 1, 1), 0) * (2 * j)
        wd = want_desc_fn(r0)
        mx = jnp.maximum(top, bot)
        mn = jnp.minimum(top, bot)
        new_top = jnp.where(wd, mx, mn)
        new_bot = jnp.where(wd, mn, mx)
        return jnp.stack([new_top, new_bot], axis=1).reshape(s, lanes)
    else:
        r = jax.lax.broadcasted_iota(jnp.int32, (s, 1), 0)
        bitj = (r & j) != 0
        up = jnp.roll(x, -j, axis=0)
        dn = jnp.roll(x, j, axis=0)
        partner = jnp.where(bitj, dn, up)
        take_max = want_desc_fn(r) ^ bitj
        mx = jnp.maximum(x, partner)
        mn = jnp.minimum(x, partner)
        return jnp.where(take_max, mx, mn)


def _truncate(x):
    s, lanes = x.shape
    xr = x.reshape(s // 128, 2, 64, lanes)
    return jnp.maximum(xr[:, 0], xr[:, 1]).reshape(s // 2, lanes)


def _topk_body(x_ref, o_ref):
    x = x_ref[0]

    for k in (2, 4, 8, 16, 32, 64):
        wd = lambda r, k=k: ((r & (k & 63)) == 0) ^ ((r & 64) != 0)
        j = k // 2
        while j >= 1:
            x = _stage(x, j, wd)
            j //= 2

    merge_wd = lambda r: (r & 64) == 0
    for _ in range(7):
        x = _truncate(x)
        for j in (32, 16, 8, 4, 2, 1):
            x = _stage(x, j, merge_wd)

    o_ref[0] = x


def kernel(inputs):
    b, n, c = inputs.shape
    return pl.pallas_call(
        _topk_body,
        grid=(b,),
        in_specs=[pl.BlockSpec((1, n, c), lambda i: (i, 0, 0))],
        out_specs=pl.BlockSpec((1, 64, c), lambda i: (i, 0, 0)),
        out_shape=jax.ShapeDtypeStruct((b, 64, c), inputs.dtype),
    )(inputs)


# speedup vs baseline: 8.6403x; 1.6996x over previous
import functools
import jax
import jax.numpy as jnp
from jax import lax
from jax.experimental import pallas as pl
from jax.experimental.pallas import tpu as pltpu
from jax.experimental.pallas import tpu_sc as plsc

_B, _N, _C = 32, 8192, 128
_K = 64
_L = 16
_NW = 32
_CG = _C // _L
_TPW = _B * _CG // _NW
_RB = 2048
_NCH = _N // _RB
_CKB = 64
_CAP = 192
_TRIG = 128
_UNR = 8


def _cmpex_stage(cand, j, k, n, alt=True):
    tj = 2 * j
    u = min(j, 16)
    nsub = j // u

    def body(m, _):
        g = m // nsub
        r0 = g * tj
        base = r0 + (m % nsub) * u
        wd = (r0 & (k & 63)) == 0
        if alt:
            wd = wd != ((r0 & 64) != 0)
        for lo in range(u):
            a = cand[base + lo]
            b = cand[base + lo + j]
            mx = jnp.maximum(a, b)
            mn = jnp.minimum(a, b)
            cand[base + lo] = jnp.where(wd, mx, mn)
            cand[base + lo + j] = jnp.where(wd, mn, mx)
        return 0

    lax.fori_loop(0, (n // tj) * nsub, body, 0)


def _sort128(cand):
    for k in (2, 4, 8, 16, 32, 64):
        j = k // 2
        while j >= 1:
            _cmpex_stage(cand, j, k, 128)
            j //= 2

    def fold(m, _):
        for lo in range(16):
            i = m * 16 + lo
            cand[i] = jnp.maximum(cand[i], cand[i + 64])
        return 0

    lax.fori_loop(0, 4, fold, 0)
    for j in (32, 16, 8, 4, 2, 1):
        _cmpex_stage(cand, j, 64, 64)


def _refil(cand, lo, hi, thr, cnt, lane, ninf):
    def body(i, c2):
        v = cand[i]
        cand[i] = ninf
        m = v > thr
        plsc.store_scatter(cand, [c2, lane], v, mask=m)
        return c2 + jnp.where(m, 1, 0)

    return lax.fori_loop(lo, hi, body, cnt)


def _sc_body(x_hbm, o_hbm, buf, cand):
    cid = lax.axis_index("c")
    sid = lax.axis_index("s")
    wid = sid * 2 + cid
    lane = lax.iota(jnp.int32, _L)
    ninf = jnp.full((_L,), -jnp.inf, jnp.float32)
    zero = jnp.zeros((_L,), jnp.int32)

    def compact(tc):
        thr, cnt = tc
        _sort128(cand)
        thr2 = jnp.maximum(thr, cand[63])
        cnt2 = _refil(cand, 0, 64, thr2, zero, lane, ninf)

        def clear(i, _):
            cand[64 + i] = ninf
            return 0

        lax.fori_loop(0, 64, clear, 0)
        cnt2 = _refil(cand, 128, _CAP, thr2, cnt2, lane, ninf)
        return thr2, cnt2

    def task_body(t, _):
        task = wid * _TPW + t
        b = task // _CG
        c0 = (task % _CG) * _L

        def init_row(i, _):
            cand[i] = ninf
            return 0

        lax.fori_loop(0, _CAP, init_row, 0)

        def block_body(blk, carry):
            def row_group(g, c2):
                thr2, cnt2 = c2
                rbase = blk * _CKB + g * _UNR
                for u in range(_UNR):
                    v = buf[rbase + u]
                    m = v > thr2
                    plsc.store_scatter(cand, [cnt2, lane], v, mask=m)
                    cnt2 = cnt2 + jnp.where(m, 1, 0)
                return thr2, cnt2

            carry = lax.fori_loop(0, _CKB // _UNR, row_group, carry)
            thr, cnt = carry
            return lax.cond(jnp.max(cnt) >= _TRIG, compact,
                            lambda tc: tc, (thr, cnt))

        def chunk_body(ch, carry):
            pltpu.sync_copy(
                x_hbm.at[b, pl.ds(ch * _RB, _RB), pl.ds(c0, _L)], buf)
            return lax.fori_loop(0, _RB // _CKB, block_body, carry)

        thr, cnt = lax.fori_loop(0, _NCH, chunk_body, (ninf, zero))

        thr, cnt = compact((thr, cnt))
        _sort128(cand)

        def fill_row(i, _):
            cand[64 + i] = thr
            return 0

        lax.fori_loop(0, _K, fill_row, 0)
        for j in (64, 32, 16, 8, 4, 2, 1):
            _cmpex_stage(cand, j, 64, 128, alt=False)

        pltpu.sync_copy(cand.at[pl.ds(0, _K)],
                        o_hbm.at[b, slice(None), pl.ds(c0, _L)])
        return 0

    lax.fori_loop(0, _TPW, task_body, 0)


@functools.cache
def _sc_topk():
    return pl.kernel(
        _sc_body,
        out_type=jax.ShapeDtypeStruct((_B, _K, _C), jnp.float32),
        mesh=plsc.VectorSubcoreMesh(
            core_axis_name="c", subcore_axis_name="s",
            num_cores=2, num_subcores=16),
        compiler_params=pltpu.CompilerParams(
            use_tc_tiling_on_sc=False, needs_layout_passes=False),
        scratch_types=[
            pltpu.VMEM((_RB, _L), jnp.float32),
            pltpu.VMEM((_CAP, _L), jnp.float32),
        ],
    )


def kernel(inputs):
    return _sc_topk()(inputs)
